# baseline (speedup 1.0000x reference)
import functools

import jax
import jax.numpy as jnp
from jax import lax
from jax.experimental import pallas as pl
from jax.experimental.pallas import tpu as pltpu
from jax.experimental.pallas import tpu_sc as plsc

_TPAD = 1024


@functools.lru_cache(maxsize=None)
def _make_lookup(n_atoms: int, n_per_atom: int, embed_dim: int):
    info = plsc.get_sparse_core_info()
    nl = info.num_lanes
    nw = info.num_cores * info.num_subcores
    assert n_atoms % nw == 0
    lanes = n_atoms // nw
    assert lanes % nl == 0 and n_per_atom % 2 == 0

    mesh = plsc.VectorSubcoreMesh(core_axis_name="c", subcore_axis_name="s")

    @functools.partial(
        pl.kernel,
        mesh=mesh,
        out_type=jax.ShapeDtypeStruct(
            (n_per_atom, embed_dim, n_atoms), jnp.float32
        ),
        scratch_types=[
            pltpu.VMEM((embed_dim * _TPAD,), jnp.float32),
            pltpu.VMEM((n_per_atom, lanes), jnp.int32),
            pltpu.VMEM((embed_dim, lanes), jnp.float32),
            pltpu.VMEM((embed_dim, lanes), jnp.float32),
            pltpu.SemaphoreType.DMA,
            pltpu.SemaphoreType.DMA,
            pltpu.SemaphoreType.DMA,
        ],
        compiler_params=pltpu.CompilerParams(
            use_tc_tiling_on_sc=False, needs_layout_passes=False
        ),
    )
    def k(tt_hbm, idxt_hbm, out_hbm, tt_v, idx_v, st0, st1, gt, s0, s1):
        bufs = ((st0, s0), (st1, s1))
        wid = lax.axis_index("s") * info.num_cores + lax.axis_index("c")
        lane0 = wid * lanes

        pltpu.async_copy(tt_hbm, tt_v, gt).wait()
        pltpu.sync_copy(idxt_hbm.at[:, pl.ds(lane0, lanes)], idx_v)

        def fill(j, p):
            st = bufs[p][0]
            for c in range(lanes // nl):
                addr = idx_v[j, pl.ds(c * nl, nl)]
                for kk in range(embed_dim):
                    st[kk, pl.ds(c * nl, nl)] = plsc.load_gather(
                        tt_v, [addr + (kk * _TPAD)]
                    )

        def scat(j, p):
            st, sc = bufs[p]
            return pltpu.make_async_copy(
                st, out_hbm.at[j, :, pl.ds(lane0, lanes)], sc
            )

        fill(0, 0)
        scat(0, 0).start()
        fill(1, 1)
        scat(1, 1).start()

        def body(i, _):
            j0 = 2 * i
            scat(j0 - 2, 0).wait()
            fill(j0, 0)
            scat(j0, 0).start()
            scat(j0 - 1, 1).wait()
            fill(j0 + 1, 1)
            scat(j0 + 1, 1).start()
            return 0

        lax.fori_loop(1, n_per_atom // 2, body, 0)
        scat(n_per_atom - 2, 0).wait()
        scat(n_per_atom - 1, 1).wait()

    return k


def kernel(atype, table):
    b0, b1 = atype.shape
    n_rows, embed_dim = table.shape
    assert n_rows <= _TPAD
    idx_t = atype.astype(jnp.int32).T
    table_t = jnp.pad(table.T, ((0, 0), (0, _TPAD - n_rows))).reshape(-1)
    y = _make_lookup(b0, b1, embed_dim)(table_t, idx_t)
    return jnp.transpose(y, (2, 0, 1))


# speedup vs baseline: 1.2955x; 1.2955x over previous
import functools

import jax
import jax.numpy as jnp
from jax import lax
from jax.experimental import pallas as pl
from jax.experimental.pallas import tpu as pltpu
from jax.experimental.pallas import tpu_sc as plsc

_TPAD = 1024


@functools.lru_cache(maxsize=None)
def _make_lookup(n_atoms: int, n_per_atom: int, embed_dim: int):
    info = plsc.get_sparse_core_info()
    nl = info.num_lanes
    nw = info.num_cores * info.num_subcores
    assert n_atoms % nw == 0
    lanes = n_atoms // nw
    assert lanes % nl == 0 and n_per_atom % 2 == 0

    mesh = plsc.VectorSubcoreMesh(core_axis_name="c", subcore_axis_name="s")

    @functools.partial(
        pl.kernel,
        mesh=mesh,
        out_type=jax.ShapeDtypeStruct(
            (n_per_atom, embed_dim // 8, nw, 8, lanes), jnp.float32
        ),
        scratch_types=[
            pltpu.VMEM((embed_dim * _TPAD,), jnp.float32),
            pltpu.VMEM((n_per_atom, lanes), jnp.int32),
            pltpu.VMEM((embed_dim // 8, 8, lanes), jnp.float32),
            pltpu.VMEM((embed_dim // 8, 8, lanes), jnp.float32),
            pltpu.SemaphoreType.DMA,
            pltpu.SemaphoreType.DMA,
            pltpu.SemaphoreType.DMA,
        ],
        compiler_params=pltpu.CompilerParams(
            use_tc_tiling_on_sc=False, needs_layout_passes=False
        ),
    )
    def k(tt_hbm, idxt_hbm, out_hbm, tt_v, idx_v, st0, st1, gt, s0, s1):
        bufs = ((st0, s0), (st1, s1))
        wid = lax.axis_index("s") * info.num_cores + lax.axis_index("c")
        lane0 = wid * lanes

        pltpu.async_copy(tt_hbm, tt_v, gt).wait()
        pltpu.sync_copy(idxt_hbm.at[:, pl.ds(lane0, lanes)], idx_v)

        def fill(j, p):
            st = bufs[p][0]
            for c in range(lanes // nl):
                addr = idx_v[j, pl.ds(c * nl, nl)]
                for kk in range(embed_dim):
                    st[kk // 8, kk % 8, pl.ds(c * nl, nl)] = plsc.load_gather(
                        tt_v, [addr]
                    )
                    addr = addr + _TPAD

        def scat(j, p):
            st, sc = bufs[p]
            return pltpu.make_async_copy(st, out_hbm.at[j, :, wid], sc)

        fill(0, 0)
        scat(0, 0).start()
        fill(1, 1)
        scat(1, 1).start()

        def body(i, _):
            j0 = 2 * i
            scat(j0 - 2, 0).wait()
            fill(j0, 0)
            scat(j0, 0).start()
            scat(j0 - 1, 1).wait()
            fill(j0 + 1, 1)
            scat(j0 + 1, 1).start()
            return 0

        lax.fori_loop(1, n_per_atom // 2, body, 0)
        scat(n_per_atom - 2, 0).wait()
        scat(n_per_atom - 1, 1).wait()

    return k


def kernel(atype, table):
    b0, b1 = atype.shape
    n_rows, embed_dim = table.shape
    assert n_rows <= _TPAD
    idx_t = atype.astype(jnp.int32).T
    table_t = jnp.pad(table.T, ((0, 0), (0, _TPAD - n_rows))).reshape(-1)
    y = _make_lookup(b0, b1, embed_dim)(table_t, idx_t)
    return jnp.transpose(y, (2, 4, 0, 1, 3)).reshape(b0, b1, embed_dim)


# speedup vs baseline: 1.3958x; 1.0774x over previous
import functools

import jax
import jax.numpy as jnp
from jax import lax
from jax.experimental import pallas as pl
from jax.experimental.pallas import tpu as pltpu
from jax.experimental.pallas import tpu_sc as plsc

_MAXG = 128


@functools.lru_cache(maxsize=None)
def _make_lookup(n_rows: int, n_atoms: int, n_per_atom: int, embed_dim: int):
    info = plsc.get_sparse_core_info()
    nw = info.num_cores * info.num_subcores
    assert n_atoms % (2 * nw) == 0
    atoms_per_w = n_atoms // nw
    segs = []
    off = 0
    while off < n_per_atom:
        n = min(_MAXG, n_per_atom - off)
        segs.append((off, n))
        off += n

    mesh = plsc.VectorSubcoreMesh(core_axis_name="c", subcore_axis_name="s")

    @functools.partial(
        pl.kernel,
        mesh=mesh,
        out_type=jax.ShapeDtypeStruct(
            (n_atoms, n_per_atom, embed_dim), jnp.float32
        ),
        scratch_types=[
            pltpu.VMEM((atoms_per_w, n_per_atom), jnp.int32),
            pltpu.VMEM((n_per_atom, embed_dim), jnp.float32),
            pltpu.VMEM((n_per_atom, embed_dim), jnp.float32),
            pltpu.VMEM_SHARED((n_rows, embed_dim), jnp.float32),
            pltpu.SemaphoreType.DMA,
            pltpu.SemaphoreType.DMA,
            pltpu.SemaphoreType.DMA,
            pltpu.SemaphoreType.DMA,
        ],
        compiler_params=pltpu.CompilerParams(use_tc_tiling_on_sc=False),
    )
    def k(table_hbm, idx_hbm, out_hbm, idx_v, st0, st1, table_sp, g0, g1, s0, s1):
        bufs = ((st0, g0, s0), (st1, g1, s1))
        sid = lax.axis_index("s")
        wid = sid * info.num_cores + lax.axis_index("c")
        a_base = wid * atoms_per_w

        @pl.when(sid == 0)
        def _():
            pltpu.sync_copy(table_hbm, table_sp)

        pltpu.sync_copy(idx_hbm.at[pl.ds(a_base, atoms_per_w)], idx_v)
        plsc.subcore_barrier()

        def gathers(a, p):
            st, sg, _ = bufs[p]
            return [
                pltpu.make_async_copy(
                    table_sp.at[idx_v.at[a, pl.ds(off, n)]],
                    st.at[pl.ds(off, n)],
                    sg,
                )
                for off, n in segs
            ]

        def scat(a, p):
            st, _, sc = bufs[p]
            return pltpu.make_async_copy(st, out_hbm.at[a_base + a], sc)

        for c in gathers(0, 0):
            c.start()
        for c in gathers(1, 1):
            c.start()
        for c in gathers(0, 0):
            c.wait()
        scat(0, 0).start()
        for c in gathers(1, 1):
            c.wait()
        scat(1, 1).start()

        def body(i, _):
            a0 = 2 * i
            scat(a0 - 2, 0).wait()
            for c in gathers(a0, 0):
                c.start()
            scat(a0 - 1, 1).wait()
            for c in gathers(a0 + 1, 1):
                c.start()
            for c in gathers(a0, 0):
                c.wait()
            scat(a0, 0).start()
            for c in gathers(a0 + 1, 1):
                c.wait()
            scat(a0 + 1, 1).start()
            return 0

        lax.fori_loop(1, atoms_per_w // 2, body, 0)
        scat(atoms_per_w - 2, 0).wait()
        scat(atoms_per_w - 1, 1).wait()

    return k


def kernel(atype, table):
    b0, b1 = atype.shape
    n_rows, embed_dim = table.shape
    idx = atype.astype(jnp.int32)
    return _make_lookup(n_rows, b0, b1, embed_dim)(table, idx)
